# Initial kernel scaffold; baseline (speedup 1.0000x reference)
import jax
import jax.numpy as jnp
from jax.experimental import pallas as pl


def kernel(x, edge_index1, edge_index2, edge_weight1, edge_weight2, pos_edge_index, W1, W2, W3):
    raise NotImplementedError("write your pallas kernel here")


# speedup vs baseline: 4.2294x; 4.2294x over previous
import functools

import jax
import jax.numpy as jnp
from jax import lax
from jax.experimental import pallas as pl
from jax.experimental.pallas import tpu as pltpu
from jax.experimental.pallas import tpu_sc as plsc

N_NODES = 10000
N_EDGES = 320000
NFEAT = 128

NC, NS, L = 2, 16, 16
NW = NC * NS
C = 80
CHUNKS_PER_W = N_EDGES // (NW * C)
RBLK = 80
NBLK = N_NODES // RBLK
GROUPS = NFEAT // L

BM = 1000


def _sc_mesh():
    return plsc.VectorSubcoreMesh(core_axis_name="c", subcore_axis_name="s")


def _edge_aggregate(table, src, dst, w):
    @functools.partial(
        pl.kernel,
        out_type=jax.ShapeDtypeStruct((NC, N_NODES, NFEAT), jnp.float32),
        mesh=_sc_mesh(),
        scratch_types=[
            pltpu.VMEM((C,), jnp.int32),
            pltpu.VMEM((C,), jnp.int32),
            pltpu.VMEM((C,), jnp.float32),
            pltpu.VMEM((C, NFEAT), jnp.float32),
            pltpu.VMEM((RBLK, NFEAT), jnp.float32),
            pltpu.VMEM_SHARED((N_NODES, NFEAT), jnp.float32),
            pltpu.SemaphoreType.DMA,
        ],
        compiler_params=pltpu.CompilerParams(needs_layout_passes=False),
    )
    def agg(table_hbm, src_hbm, dst_hbm, w_hbm, out_hbm,
            src_v, dst_v, w_v, rows_v, zbuf, acc, sem):
        cid = lax.axis_index("c")
        sid = lax.axis_index("s")
        wid = cid * NS + sid

        def zfill(i, carry):
            for g in range(GROUPS):
                zbuf[i, pl.ds(g * L, L)] = jnp.zeros((L,), jnp.float32)
            return carry
        lax.fori_loop(0, RBLK, zfill, 0)
        for k in range(NBLK // NS + 1):
            b = sid + NS * k
            @pl.when(b < NBLK)
            def _():
                pltpu.sync_copy(zbuf, acc.at[pl.ds(b * RBLK, RBLK)])
        plsc.subcore_barrier()

        def chunk_body(j, carry):
            base = (j * NW + wid) * C
            pltpu.sync_copy(src_hbm.at[pl.ds(base, C)], src_v)
            pltpu.sync_copy(dst_hbm.at[pl.ds(base, C)], dst_v)
            pltpu.sync_copy(w_hbm.at[pl.ds(base, C)], w_v)
            pltpu.async_copy(table_hbm.at[src_v], rows_v, sem).wait()

            def scale(e, c2):
                wb = plsc.load_gather(w_v, [jnp.full((L,), e, jnp.int32)])
                for g in range(GROUPS):
                    rows_v[e, pl.ds(g * L, L)] = rows_v[e, pl.ds(g * L, L)] * wb
                return c2
            lax.fori_loop(0, C, scale, 0)

            pltpu.sync_copy(rows_v, acc.at[dst_v], add=True)
            return carry
        lax.fori_loop(0, CHUNKS_PER_W, chunk_body, 0)

        plsc.subcore_barrier()
        for k in range(NBLK // NS + 1):
            b = sid + NS * k
            @pl.when(b < NBLK)
            def _():
                r0 = b * RBLK
                pltpu.sync_copy(acc.at[pl.ds(r0, RBLK)],
                                out_hbm.at[cid, pl.ds(r0, RBLK)])

    return agg(table, src, dst, w)


def _mm_relu(P, W):
    def body(p_ref, w_ref, o_ref):
        s = p_ref[0] + p_ref[1]
        o_ref[...] = jnp.maximum(
            jnp.dot(s, w_ref[...], preferred_element_type=jnp.float32), 0.0)

    return pl.pallas_call(
        body,
        grid=(N_NODES // BM,),
        in_specs=[pl.BlockSpec((NC, BM, NFEAT), lambda i: (0, i, 0)),
                  pl.BlockSpec((NFEAT, NFEAT), lambda i: (0, 0))],
        out_specs=pl.BlockSpec((BM, NFEAT), lambda i: (i, 0)),
        out_shape=jax.ShapeDtypeStruct((N_NODES, NFEAT), jnp.float32),
    )(P, W)


def _mm_uv(Q, W2, W3):
    def body(q_ref, w2_ref, w3_ref, o_ref):
        w3r = jnp.concatenate([w3_ref[0:NFEAT, :], w3_ref[NFEAT:, :]], axis=1)
        w23 = jnp.dot(w2_ref[...], w3r, preferred_element_type=jnp.float32)
        s = q_ref[0] + q_ref[1]
        o_ref[...] = jnp.dot(s, w23, preferred_element_type=jnp.float32)

    return pl.pallas_call(
        body,
        grid=(N_NODES // BM,),
        in_specs=[pl.BlockSpec((NC, BM, NFEAT), lambda i: (0, i, 0)),
                  pl.BlockSpec((NFEAT, NFEAT), lambda i: (0, 0)),
                  pl.BlockSpec((2 * NFEAT, 2), lambda i: (0, 0))],
        out_specs=pl.BlockSpec((BM, 4), lambda i: (i, 0)),
        out_shape=jax.ShapeDtypeStruct((N_NODES, 4), jnp.float32),
    )(Q, W2, W3)


def _decode(uvf, ps, pd):
    @functools.partial(
        pl.kernel,
        out_type=(jax.ShapeDtypeStruct((N_EDGES,), jnp.float32),
                  jax.ShapeDtypeStruct((N_EDGES,), jnp.float32)),
        mesh=_sc_mesh(),
        scratch_types=[
            pltpu.VMEM((N_NODES * 4,), jnp.float32),
            pltpu.VMEM((C,), jnp.int32),
            pltpu.VMEM((C,), jnp.int32),
            pltpu.VMEM((C,), jnp.float32),
            pltpu.VMEM((C,), jnp.float32),
        ],
        compiler_params=pltpu.CompilerParams(needs_layout_passes=False),
    )
    def dec(uv_hbm, ps_hbm, pd_hbm, o0_hbm, o1_hbm, uv_v, s_v, d_v, ob0, ob1):
        cid = lax.axis_index("c")
        sid = lax.axis_index("s")
        wid = cid * NS + sid
        pltpu.sync_copy(uv_hbm, uv_v)

        def chunk_body(j, carry):
            base = (j * NW + wid) * C
            pltpu.sync_copy(ps_hbm.at[pl.ds(base, C)], s_v)
            pltpu.sync_copy(pd_hbm.at[pl.ds(base, C)], d_v)
            for g in range(C // L):
                si = s_v[pl.ds(g * L, L)] * 4
                di = d_v[pl.ds(g * L, L)] * 4
                u0 = plsc.load_gather(uv_v, [si])
                u1 = plsc.load_gather(uv_v, [si + 1])
                v0 = plsc.load_gather(uv_v, [di + 2])
                v1 = plsc.load_gather(uv_v, [di + 3])
                ob0[pl.ds(g * L, L)] = u0 + v0
                ob1[pl.ds(g * L, L)] = u1 + v1
            pltpu.sync_copy(ob0, o0_hbm.at[pl.ds(base, C)])
            pltpu.sync_copy(ob1, o1_hbm.at[pl.ds(base, C)])
            return carry
        lax.fori_loop(0, CHUNKS_PER_W, chunk_body, 0)

    o0, o1 = dec(uvf, ps, pd)
    return jnp.stack([o0, o1], axis=1)


def kernel(x, edge_index1, edge_index2, edge_weight1, edge_weight2,
           pos_edge_index, W1, W2, W3):
    src1 = edge_index1[0].astype(jnp.int32)
    dst1 = edge_index1[1].astype(jnp.int32)
    src2 = edge_index2[0].astype(jnp.int32)
    dst2 = edge_index2[1].astype(jnp.int32)
    ps = pos_edge_index[0].astype(jnp.int32)
    pd = pos_edge_index[1].astype(jnp.int32)

    P1 = _edge_aggregate(x.astype(jnp.float32), src1, dst1, edge_weight1)
    h = _mm_relu(P1, W1)
    P2 = _edge_aggregate(h, src2, dst2, edge_weight2)
    uv = _mm_uv(P2, W2, W3)
    return _decode(uv.reshape(-1), ps, pd)


# speedup vs baseline: 8.2201x; 1.9436x over previous
import functools

import jax
import jax.numpy as jnp
from jax import lax
from jax.experimental import pallas as pl
from jax.experimental.pallas import tpu as pltpu
from jax.experimental.pallas import tpu_sc as plsc

N_NODES = 10000
N_EDGES = 320000
NFEAT = 128

NC, NS, L = 2, 16, 16
NW = NC * NS
C = 80
CHUNKS_PER_W = N_EDGES // (NW * C)
RBLK = 80
NBLK = N_NODES // RBLK
GROUPS = NFEAT // L

BM = 1000


def _sc_mesh():
    return plsc.VectorSubcoreMesh(core_axis_name="c", subcore_axis_name="s")


def _pack_edges(edge_index, w):
    src = edge_index[0].astype(jnp.int32).reshape(-1, C)
    wbits = jax.lax.bitcast_convert_type(w.astype(jnp.float32), jnp.int32)
    sw = jnp.concatenate([src, wbits.reshape(-1, C)], axis=1).reshape(-1)
    return sw, edge_index[1].astype(jnp.int32)


def _edge_aggregate(table, sw, dstf):
    C2 = 2 * C
    LAST = CHUNKS_PER_W - 1

    @functools.partial(
        pl.kernel,
        out_type=jax.ShapeDtypeStruct((NC, N_NODES, NFEAT), jnp.float32),
        mesh=_sc_mesh(),
        scratch_types=[
            pltpu.VMEM((2 * C2,), jnp.int32),
            pltpu.VMEM((4, C), jnp.int32),
            pltpu.VMEM((2, C, NFEAT), jnp.float32),
            pltpu.VMEM_SHARED((N_NODES, NFEAT), jnp.float32),
            pltpu.SemaphoreType.DMA,
            pltpu.SemaphoreType.DMA,
            pltpu.SemaphoreType.DMA,
            pltpu.SemaphoreType.DMA,
            pltpu.SemaphoreType.DMA,
            pltpu.SemaphoreType.DMA,
        ],
        compiler_params=pltpu.CompilerParams(needs_layout_passes=False),
    )
    def agg(table_hbm, sw_hbm, dst_hbm, out_hbm,
            sw2, dst4, rows2, acc, semI0, semI1, semG0, semG1, semS0, semS1):
        cid = lax.axis_index("c")
        sid = lax.axis_index("s")
        wid = cid * NS + sid
        semI = (semI0, semI1)
        semG = (semG0, semG1)
        semS = (semS0, semS1)

        def zfill(i, carry):
            for g in range(GROUPS):
                rows2[0, i, pl.ds(g * L, L)] = jnp.zeros((L,), jnp.float32)
            return carry
        lax.fori_loop(0, C, zfill, 0)
        for k in range(NBLK // NS + 1):
            b = sid + NS * k
            @pl.when(b < NBLK)
            def _():
                pltpu.sync_copy(rows2.at[0], acc.at[pl.ds(b * RBLK, RBLK)])
        plsc.subcore_barrier()

        def issue_idx(j, s, d):
            g = j * NW + wid
            pltpu.async_copy(sw_hbm.at[pl.ds(g * C2, C2)],
                             sw2.at[pl.ds(s * C2, C2)], semI[s])
            pltpu.async_copy(dst_hbm.at[pl.ds(g * C, C)], dst4.at[d], semI[s])

        def wait_idx(s, d):
            pltpu.make_async_copy(sw_hbm.at[pl.ds(0, C2)],
                                  sw2.at[pl.ds(s * C2, C2)], semI[s]).wait()
            pltpu.make_async_copy(dst_hbm.at[pl.ds(0, C)], dst4.at[d],
                                  semI[s]).wait()

        def issue_gather(s):
            pltpu.async_copy(table_hbm.at[sw2.at[pl.ds(s * C2, C)]],
                             rows2.at[s], semG[s])

        def wait_gather(s):
            pltpu.make_async_copy(table_hbm.at[sw2.at[pl.ds(s * C2, C)]],
                                  rows2.at[s], semG[s]).wait()

        def scale(s):
            def body(e, carry):
                wb = plsc.bitcast(
                    plsc.load_gather(
                        sw2, [jnp.full((L,), s * C2 + C, jnp.int32) + e]),
                    jnp.float32)
                for g in range(GROUPS):
                    rows2[s, e, pl.ds(g * L, L)] = rows2[s, e, pl.ds(g * L, L)] * wb
                return carry
            lax.fori_loop(0, C, body, 0)

        def issue_scatter(s, d):
            pltpu.async_copy(rows2.at[s], acc.at[dst4.at[d]], semS[s], add=True)

        def wait_scatter(s, d):
            pltpu.make_async_copy(rows2.at[s], acc.at[dst4.at[d]],
                                  semS[s]).wait()

        def step(j, c, first=False, prefetch=True, fetch_next=True):
            s, o, d = c % 2, 1 - c % 2, c % 4
            if not first:
                wait_scatter(o, (c - 1) % 4)
            if fetch_next:
                wait_idx(o, (c + 1) % 4)
                issue_gather(o)
            wait_gather(s)
            scale(s)
            issue_scatter(s, d)
            if prefetch:
                issue_idx(j + 2, s, (c + 2) % 4)

        issue_idx(0, 0, 0)
        issue_idx(1, 1, 1)
        wait_idx(0, 0)
        issue_gather(0)
        step(0, 0, first=True)
        step(1, 1)
        step(2, 2)
        step(3, 3)

        def pair4(jj, carry):
            j0 = 4 * jj
            step(j0 + 0, 0)
            step(j0 + 1, 1)
            step(j0 + 2, 2)
            step(j0 + 3, 3)
            return carry
        lax.fori_loop(1, 30, pair4, 0)

        step(120, 0)
        step(121, 1)
        step(122, 2)
        step(123, 3, prefetch=False)
        step(124, 0, prefetch=False, fetch_next=False)
        wait_scatter(0, 0)

        plsc.subcore_barrier()
        for k in range(NBLK // NS + 1):
            b = sid + NS * k
            @pl.when(b < NBLK)
            def _():
                r0 = b * RBLK
                pltpu.sync_copy(acc.at[pl.ds(r0, RBLK)],
                                out_hbm.at[cid, pl.ds(r0, RBLK)])

    return agg(table, sw, dstf)


def _mm_relu(P, W):
    def body(p_ref, w_ref, o_ref):
        s = p_ref[0] + p_ref[1]
        o_ref[...] = jnp.maximum(
            jnp.dot(s, w_ref[...], preferred_element_type=jnp.float32), 0.0)

    return pl.pallas_call(
        body,
        grid=(N_NODES // BM,),
        in_specs=[pl.BlockSpec((NC, BM, NFEAT), lambda i: (0, i, 0)),
                  pl.BlockSpec((NFEAT, NFEAT), lambda i: (0, 0))],
        out_specs=pl.BlockSpec((BM, NFEAT), lambda i: (i, 0)),
        out_shape=jax.ShapeDtypeStruct((N_NODES, NFEAT), jnp.float32),
    )(P, W)


def _mm_uv(Q, W2, W3):
    def body(q_ref, w2_ref, w3_ref, o_ref):
        w3r = jnp.concatenate([w3_ref[0:NFEAT, :], w3_ref[NFEAT:, :]], axis=1)
        w23 = jnp.dot(w2_ref[...], w3r, preferred_element_type=jnp.float32)
        s = q_ref[0] + q_ref[1]
        o_ref[...] = jnp.dot(s, w23, preferred_element_type=jnp.float32)

    return pl.pallas_call(
        body,
        grid=(N_NODES // BM,),
        in_specs=[pl.BlockSpec((NC, BM, NFEAT), lambda i: (0, i, 0)),
                  pl.BlockSpec((NFEAT, NFEAT), lambda i: (0, 0)),
                  pl.BlockSpec((2 * NFEAT, 2), lambda i: (0, 0))],
        out_specs=pl.BlockSpec((BM, 4), lambda i: (i, 0)),
        out_shape=jax.ShapeDtypeStruct((N_NODES, 4), jnp.float32),
    )(Q, W2, W3)


def _decode(uvf, ps, pd):
    @functools.partial(
        pl.kernel,
        out_type=(jax.ShapeDtypeStruct((N_EDGES,), jnp.float32),
                  jax.ShapeDtypeStruct((N_EDGES,), jnp.float32)),
        mesh=_sc_mesh(),
        scratch_types=[
            pltpu.VMEM((N_NODES * 4,), jnp.float32),
            pltpu.VMEM((C,), jnp.int32),
            pltpu.VMEM((C,), jnp.int32),
            pltpu.VMEM((C,), jnp.float32),
            pltpu.VMEM((C,), jnp.float32),
        ],
        compiler_params=pltpu.CompilerParams(needs_layout_passes=False),
    )
    def dec(uv_hbm, ps_hbm, pd_hbm, o0_hbm, o1_hbm, uv_v, s_v, d_v, ob0, ob1):
        cid = lax.axis_index("c")
        sid = lax.axis_index("s")
        wid = cid * NS + sid
        pltpu.sync_copy(uv_hbm, uv_v)

        def chunk_body(j, carry):
            base = (j * NW + wid) * C
            pltpu.sync_copy(ps_hbm.at[pl.ds(base, C)], s_v)
            pltpu.sync_copy(pd_hbm.at[pl.ds(base, C)], d_v)
            for g in range(C // L):
                si = s_v[pl.ds(g * L, L)] * 4
                di = d_v[pl.ds(g * L, L)] * 4
                u0 = plsc.load_gather(uv_v, [si])
                u1 = plsc.load_gather(uv_v, [si + 1])
                v0 = plsc.load_gather(uv_v, [di + 2])
                v1 = plsc.load_gather(uv_v, [di + 3])
                ob0[pl.ds(g * L, L)] = u0 + v0
                ob1[pl.ds(g * L, L)] = u1 + v1
            pltpu.sync_copy(ob0, o0_hbm.at[pl.ds(base, C)])
            pltpu.sync_copy(ob1, o1_hbm.at[pl.ds(base, C)])
            return carry
        lax.fori_loop(0, CHUNKS_PER_W, chunk_body, 0)

    o0, o1 = dec(uvf, ps, pd)
    return jnp.stack([o0, o1], axis=1)


def kernel(x, edge_index1, edge_index2, edge_weight1, edge_weight2,
           pos_edge_index, W1, W2, W3):
    sw1, dst1 = _pack_edges(edge_index1, edge_weight1)
    sw2_, dst2 = _pack_edges(edge_index2, edge_weight2)
    ps = pos_edge_index[0].astype(jnp.int32)
    pd = pos_edge_index[1].astype(jnp.int32)

    P1 = _edge_aggregate(x.astype(jnp.float32), sw1, dst1)
    h = _mm_relu(P1, W1)
    P2 = _edge_aggregate(h, sw2_, dst2)
    uv = _mm_uv(P2, W2, W3)
    return _decode(uv.reshape(-1), ps, pd)


# speedup vs baseline: 9.8797x; 1.2019x over previous
import functools

import jax
import jax.numpy as jnp
from jax import lax
from jax.experimental import pallas as pl
from jax.experimental.pallas import tpu as pltpu
from jax.experimental.pallas import tpu_sc as plsc

N_NODES = 10000
N_EDGES = 320000
NFEAT = 128

NC, NS, L = 2, 16, 16
NW = NC * NS
C = 80
CHUNKS_PER_W = N_EDGES // (NW * C)
RBLK = 80
NBLK = N_NODES // RBLK
GROUPS = NFEAT // L

BM = 1000


def _sc_mesh():
    return plsc.VectorSubcoreMesh(core_axis_name="c", subcore_axis_name="s")


def _pack_edges(edge_index, w):
    src = edge_index[0].astype(jnp.int32).reshape(-1, C)
    wbits = jax.lax.bitcast_convert_type(w.astype(jnp.float32), jnp.int32)
    sw = jnp.concatenate([src, wbits.reshape(-1, C)], axis=1).reshape(-1)
    return sw, edge_index[1].astype(jnp.int32)


def _edge_aggregate(table, sw, dstf):
    C2 = 2 * C
    LAST = CHUNKS_PER_W - 1

    @functools.partial(
        pl.kernel,
        out_type=jax.ShapeDtypeStruct((NC, N_NODES, NFEAT), jnp.float32),
        mesh=_sc_mesh(),
        scratch_types=[
            pltpu.VMEM((2 * C2,), jnp.int32),
            pltpu.VMEM((4, C), jnp.int32),
            pltpu.VMEM((4, C, NFEAT), jnp.float32),
            pltpu.VMEM_SHARED((N_NODES, NFEAT), jnp.float32),
            pltpu.SemaphoreType.DMA,
            pltpu.SemaphoreType.DMA,
            pltpu.SemaphoreType.DMA,
            pltpu.SemaphoreType.DMA,
            pltpu.SemaphoreType.DMA,
            pltpu.SemaphoreType.DMA,
            pltpu.SemaphoreType.DMA,
            pltpu.SemaphoreType.DMA,
            pltpu.SemaphoreType.DMA,
            pltpu.SemaphoreType.DMA,
        ],
        compiler_params=pltpu.CompilerParams(needs_layout_passes=False),
    )
    def agg(table_hbm, sw_hbm, dst_hbm, out_hbm,
            sw2, dst4, rows2, acc,
            semI0, semI1, semG0, semG1, semG2, semG3,
            semS0, semS1, semS2, semS3):
        cid = lax.axis_index("c")
        sid = lax.axis_index("s")
        wid = cid * NS + sid
        semI = (semI0, semI1)
        semG = (semG0, semG1, semG2, semG3)
        semS = (semS0, semS1, semS2, semS3)

        def zfill(i, carry):
            for g in range(GROUPS):
                rows2[0, i, pl.ds(g * L, L)] = jnp.zeros((L,), jnp.float32)
            return carry
        lax.fori_loop(0, C, zfill, 0)
        for k in range(NBLK // NS + 1):
            b = sid + NS * k
            @pl.when(b < NBLK)
            def _():
                pltpu.sync_copy(rows2.at[0], acc.at[pl.ds(b * RBLK, RBLK)])
        plsc.subcore_barrier()

        def issue_idx(j, s, d):
            g = j * NW + wid
            pltpu.async_copy(sw_hbm.at[pl.ds(g * C2, C2)],
                             sw2.at[pl.ds(s * C2, C2)], semI[s])
            pltpu.async_copy(dst_hbm.at[pl.ds(g * C, C)], dst4.at[d], semI[s])

        def wait_idx(s, d):
            pltpu.make_async_copy(sw_hbm.at[pl.ds(0, C2)],
                                  sw2.at[pl.ds(s * C2, C2)], semI[s]).wait()
            pltpu.make_async_copy(dst_hbm.at[pl.ds(0, C)], dst4.at[d],
                                  semI[s]).wait()

        def issue_gather(r, s):
            pltpu.async_copy(table_hbm.at[sw2.at[pl.ds(s * C2, C)]],
                             rows2.at[r], semG[r])

        def wait_gather(r, s):
            pltpu.make_async_copy(table_hbm.at[sw2.at[pl.ds(s * C2, C)]],
                                  rows2.at[r], semG[r]).wait()

        def scale(r, s):
            def body(e, carry):
                wb = plsc.bitcast(
                    plsc.load_gather(
                        sw2, [jnp.full((L,), s * C2 + C, jnp.int32) + e]),
                    jnp.float32)
                for g in range(GROUPS):
                    rows2[r, e, pl.ds(g * L, L)] = rows2[r, e, pl.ds(g * L, L)] * wb
                return carry
            lax.fori_loop(0, C, body, 0)

        def issue_scatter(r):
            pltpu.async_copy(rows2.at[r], acc.at[dst4.at[r]], semS[r], add=True)

        def wait_scatter(r):
            pltpu.make_async_copy(rows2.at[r], acc.at[dst4.at[r]],
                                  semS[r]).wait()

        def step(j, c, wait_old=True, prefetch=True, fetch_next=True):
            r, s, o = c % 4, c % 2, 1 - c % 2
            if fetch_next:
                wait_idx(o, (c + 1) % 4)
                issue_gather((c + 1) % 4, o)
            wait_gather(r, s)
            scale(r, s)
            issue_scatter(r)
            if prefetch:
                if wait_old:
                    wait_scatter((c + 2) % 4)
                issue_idx(j + 2, s, (c + 2) % 4)

        issue_idx(0, 0, 0)
        issue_idx(1, 1, 1)
        wait_idx(0, 0)
        issue_gather(0, 0)
        step(0, 0, wait_old=False)
        step(1, 1, wait_old=False)
        step(2, 2)
        step(3, 3)

        def quad(jj, carry):
            j0 = 4 * jj
            step(j0 + 0, 0)
            step(j0 + 1, 1)
            step(j0 + 2, 2)
            step(j0 + 3, 3)
            return carry
        lax.fori_loop(1, 30, quad, 0)

        step(120, 0)
        step(121, 1)
        step(122, 2)
        step(123, 3, prefetch=False)
        step(124, 0, prefetch=False, fetch_next=False)
        for r in (1, 2, 3, 0):
            wait_scatter(r)

        plsc.subcore_barrier()
        for k in range(NBLK // NS + 1):
            b = sid + NS * k
            @pl.when(b < NBLK)
            def _():
                r0 = b * RBLK
                pltpu.sync_copy(acc.at[pl.ds(r0, RBLK)],
                                out_hbm.at[cid, pl.ds(r0, RBLK)])

    return agg(table, sw, dstf)


def _mm_relu(P, W):
    def body(p_ref, w_ref, o_ref):
        s = p_ref[0] + p_ref[1]
        o_ref[...] = jnp.maximum(
            jnp.dot(s, w_ref[...], preferred_element_type=jnp.float32), 0.0)

    return pl.pallas_call(
        body,
        grid=(N_NODES // BM,),
        in_specs=[pl.BlockSpec((NC, BM, NFEAT), lambda i: (0, i, 0)),
                  pl.BlockSpec((NFEAT, NFEAT), lambda i: (0, 0))],
        out_specs=pl.BlockSpec((BM, NFEAT), lambda i: (i, 0)),
        out_shape=jax.ShapeDtypeStruct((N_NODES, NFEAT), jnp.float32),
    )(P, W)


def _mm_uv(Q, W2, W3):
    def body(q_ref, w2_ref, w3_ref, o_ref):
        w3r = jnp.concatenate([w3_ref[0:NFEAT, :], w3_ref[NFEAT:, :]], axis=1)
        w23 = jnp.dot(w2_ref[...], w3r, preferred_element_type=jnp.float32)
        s = q_ref[0] + q_ref[1]
        o_ref[...] = jnp.dot(s, w23, preferred_element_type=jnp.float32)

    return pl.pallas_call(
        body,
        grid=(N_NODES // BM,),
        in_specs=[pl.BlockSpec((NC, BM, NFEAT), lambda i: (0, i, 0)),
                  pl.BlockSpec((NFEAT, NFEAT), lambda i: (0, 0)),
                  pl.BlockSpec((2 * NFEAT, 2), lambda i: (0, 0))],
        out_specs=pl.BlockSpec((BM, 4), lambda i: (i, 0)),
        out_shape=jax.ShapeDtypeStruct((N_NODES, 4), jnp.float32),
    )(Q, W2, W3)


def _decode(uvf, pq):
    C2 = 2 * C
    LAST = CHUNKS_PER_W - 1

    @functools.partial(
        pl.kernel,
        out_type=(jax.ShapeDtypeStruct((N_EDGES,), jnp.float32),
                  jax.ShapeDtypeStruct((N_EDGES,), jnp.float32)),
        mesh=_sc_mesh(),
        scratch_types=[
            pltpu.VMEM((N_NODES * 4,), jnp.float32),
            pltpu.VMEM((2 * C2,), jnp.int32),
            pltpu.VMEM((2, 2, C), jnp.float32),
            pltpu.SemaphoreType.DMA,
            pltpu.SemaphoreType.DMA,
            pltpu.SemaphoreType.DMA,
            pltpu.SemaphoreType.DMA,
        ],
        compiler_params=pltpu.CompilerParams(needs_layout_passes=False),
    )
    def dec(uv_hbm, pq_hbm, o0_hbm, o1_hbm, uv_v, pq2, ob,
            semI0, semI1, semO0, semO1):
        cid = lax.axis_index("c")
        sid = lax.axis_index("s")
        wid = cid * NS + sid
        semI = (semI0, semI1)
        semO = (semO0, semO1)
        pltpu.sync_copy(uv_hbm, uv_v)

        def issue_idx(j, s):
            g = j * NW + wid
            pltpu.async_copy(pq_hbm.at[pl.ds(g * C2, C2)],
                             pq2.at[pl.ds(s * C2, C2)], semI[s])

        def wait_idx(s):
            pltpu.make_async_copy(pq_hbm.at[pl.ds(0, C2)],
                                  pq2.at[pl.ds(s * C2, C2)], semI[s]).wait()

        def issue_out(j, s):
            base = (j * NW + wid) * C
            pltpu.async_copy(ob.at[s, 0], o0_hbm.at[pl.ds(base, C)], semO[s])
            pltpu.async_copy(ob.at[s, 1], o1_hbm.at[pl.ds(base, C)], semO[s])

        def wait_out(s):
            pltpu.make_async_copy(ob.at[s, 0], o0_hbm.at[pl.ds(0, C)],
                                  semO[s]).wait()
            pltpu.make_async_copy(ob.at[s, 1], o1_hbm.at[pl.ds(0, C)],
                                  semO[s]).wait()

        def step(j, c, wait_old=True, prefetch=True):
            s = c % 2
            if wait_old:
                wait_out(s)
            wait_idx(s)
            for g in range(C // L):
                si = pq2[pl.ds(s * C2 + g * L, L)] * 4
                di = pq2[pl.ds(s * C2 + C + g * L, L)] * 4
                u0 = plsc.load_gather(uv_v, [si])
                u1 = plsc.load_gather(uv_v, [si + 1])
                v0 = plsc.load_gather(uv_v, [di + 2])
                v1 = plsc.load_gather(uv_v, [di + 3])
                ob[s, 0, pl.ds(g * L, L)] = u0 + v0
                ob[s, 1, pl.ds(g * L, L)] = u1 + v1
            issue_out(j, s)
            if prefetch:
                issue_idx(j + 2, s)

        issue_idx(0, 0)
        issue_idx(1, 1)
        step(0, 0, wait_old=False)
        step(1, 1, wait_old=False)

        def duo(jj, carry):
            j0 = 2 * jj
            step(j0 + 0, 0)
            step(j0 + 1, 1)
            return carry
        lax.fori_loop(1, 61, duo, 0)

        step(122, 0)
        step(123, 1, prefetch=False)
        step(124, 0, prefetch=False)
        wait_out(1)
        wait_out(0)

    o0, o1 = dec(uvf, pq)
    return jnp.stack([o0, o1], axis=1)


def kernel(x, edge_index1, edge_index2, edge_weight1, edge_weight2,
           pos_edge_index, W1, W2, W3):
    sw1, dst1 = _pack_edges(edge_index1, edge_weight1)
    sw2_, dst2 = _pack_edges(edge_index2, edge_weight2)
    pq = jnp.concatenate([pos_edge_index[0].astype(jnp.int32).reshape(-1, C),
                          pos_edge_index[1].astype(jnp.int32).reshape(-1, C)],
                         axis=1).reshape(-1)

    P1 = _edge_aggregate(x.astype(jnp.float32), sw1, dst1)
    h = _mm_relu(P1, W1)
    P2 = _edge_aggregate(h, sw2_, dst2)
    uv = _mm_uv(P2, W2, W3)
    return _decode(uv.reshape(-1), pq)


# speedup vs baseline: 10.1118x; 1.0235x over previous
import functools

import jax
import jax.numpy as jnp
from jax import lax
from jax.experimental import pallas as pl
from jax.experimental.pallas import tpu as pltpu
from jax.experimental.pallas import tpu_sc as plsc

N_NODES = 10000
N_EDGES = 320000
NFEAT = 128

NC, NS, L = 2, 16, 16
NW = NC * NS
C = 80
CHUNKS_PER_W = N_EDGES // (NW * C)
RBLK = 80
NBLK = N_NODES // RBLK
GROUPS = NFEAT // L

BM = 1000


def _sc_mesh():
    return plsc.VectorSubcoreMesh(core_axis_name="c", subcore_axis_name="s")


def _pack_edges(edge_index, w):
    src = edge_index[0].astype(jnp.int32).reshape(-1, C)
    wbits = jax.lax.bitcast_convert_type(w.astype(jnp.float32), jnp.int32)
    sw = jnp.concatenate([src, wbits.reshape(-1, C)], axis=1).reshape(-1)
    return sw, edge_index[1].astype(jnp.int32)


def _edge_aggregate(table, sw, dstf):
    C2 = 2 * C
    LAST = CHUNKS_PER_W - 1

    @functools.partial(
        pl.kernel,
        out_type=jax.ShapeDtypeStruct((NC, N_NODES, NFEAT), jnp.float32),
        mesh=_sc_mesh(),
        scratch_types=[
            pltpu.VMEM((2 * C2,), jnp.int32),
            pltpu.VMEM((4, C), jnp.int32),
            pltpu.VMEM((4, C, NFEAT), jnp.float32),
            pltpu.VMEM_SHARED((N_NODES, NFEAT), jnp.float32),
            pltpu.SemaphoreType.DMA,
            pltpu.SemaphoreType.DMA,
            pltpu.SemaphoreType.DMA,
            pltpu.SemaphoreType.DMA,
            pltpu.SemaphoreType.DMA,
            pltpu.SemaphoreType.DMA,
            pltpu.SemaphoreType.DMA,
            pltpu.SemaphoreType.DMA,
            pltpu.SemaphoreType.DMA,
            pltpu.SemaphoreType.DMA,
        ],
        compiler_params=pltpu.CompilerParams(needs_layout_passes=False),
    )
    def agg(table_hbm, sw_hbm, dst_hbm, out_hbm,
            sw2, dst4, rows2, acc,
            semI0, semI1, semG0, semG1, semG2, semG3,
            semS0, semS1, semS2, semS3):
        cid = lax.axis_index("c")
        sid = lax.axis_index("s")
        wid = cid * NS + sid
        semI = (semI0, semI1)
        semG = (semG0, semG1, semG2, semG3)
        semS = (semS0, semS1, semS2, semS3)

        def zfill(i, carry):
            for g in range(GROUPS):
                rows2[0, i, pl.ds(g * L, L)] = jnp.zeros((L,), jnp.float32)
            return carry
        lax.fori_loop(0, C, zfill, 0)
        for k in range(NBLK // NS + 1):
            b = sid + NS * k
            @pl.when(b < NBLK)
            def _():
                pltpu.sync_copy(rows2.at[0], acc.at[pl.ds(b * RBLK, RBLK)])
        plsc.subcore_barrier()

        def issue_idx(j, s, d):
            g = j * NW + wid
            pltpu.async_copy(sw_hbm.at[pl.ds(g * C2, C2)],
                             sw2.at[pl.ds(s * C2, C2)], semI[s])
            pltpu.async_copy(dst_hbm.at[pl.ds(g * C, C)], dst4.at[d], semI[s])

        def wait_idx(s, d):
            pltpu.make_async_copy(sw_hbm.at[pl.ds(0, C2)],
                                  sw2.at[pl.ds(s * C2, C2)], semI[s]).wait()
            pltpu.make_async_copy(dst_hbm.at[pl.ds(0, C)], dst4.at[d],
                                  semI[s]).wait()

        def issue_gather(r, s):
            pltpu.async_copy(table_hbm.at[sw2.at[pl.ds(s * C2, C)]],
                             rows2.at[r], semG[r])

        def wait_gather(r, s):
            pltpu.make_async_copy(table_hbm.at[sw2.at[pl.ds(s * C2, C)]],
                                  rows2.at[r], semG[r]).wait()

        def scale(r, s):
            def body(e, carry):
                wb = plsc.bitcast(
                    plsc.load_gather(
                        sw2, [jnp.full((L,), s * C2 + C, jnp.int32) + e]),
                    jnp.float32)
                for g in range(GROUPS):
                    rows2[r, e, pl.ds(g * L, L)] = rows2[r, e, pl.ds(g * L, L)] * wb
                return carry
            lax.fori_loop(0, C, body, 0, unroll=4)

        def issue_scatter(r):
            pltpu.async_copy(rows2.at[r], acc.at[dst4.at[r]], semS[r], add=True)

        def wait_scatter(r):
            pltpu.make_async_copy(rows2.at[r], acc.at[dst4.at[r]],
                                  semS[r]).wait()

        def step(j, c, wait_old=True, prefetch=True, fetch_next=True):
            r, s, o = c % 4, c % 2, 1 - c % 2
            if fetch_next:
                wait_idx(o, (c + 1) % 4)
                issue_gather((c + 1) % 4, o)
            wait_gather(r, s)
            scale(r, s)
            issue_scatter(r)
            if prefetch:
                if wait_old:
                    wait_scatter((c + 2) % 4)
                issue_idx(j + 2, s, (c + 2) % 4)

        issue_idx(0, 0, 0)
        issue_idx(1, 1, 1)
        wait_idx(0, 0)
        issue_gather(0, 0)
        step(0, 0, wait_old=False)
        step(1, 1, wait_old=False)
        step(2, 2)
        step(3, 3)

        def quad(jj, carry):
            j0 = 4 * jj
            step(j0 + 0, 0)
            step(j0 + 1, 1)
            step(j0 + 2, 2)
            step(j0 + 3, 3)
            return carry
        lax.fori_loop(1, 30, quad, 0)

        step(120, 0)
        step(121, 1)
        step(122, 2)
        step(123, 3, prefetch=False)
        step(124, 0, prefetch=False, fetch_next=False)
        for r in (1, 2, 3, 0):
            wait_scatter(r)

        plsc.subcore_barrier()
        for k in range(NBLK // NS + 1):
            b = sid + NS * k
            @pl.when(b < NBLK)
            def _():
                r0 = b * RBLK
                pltpu.sync_copy(acc.at[pl.ds(r0, RBLK)],
                                out_hbm.at[cid, pl.ds(r0, RBLK)])

    return agg(table, sw, dstf)


def _mm_relu(P, W):
    def body(p_ref, w_ref, o_ref):
        s = p_ref[0] + p_ref[1]
        o_ref[...] = jnp.maximum(
            jnp.dot(s, w_ref[...], preferred_element_type=jnp.float32), 0.0)

    return pl.pallas_call(
        body,
        grid=(N_NODES // BM,),
        in_specs=[pl.BlockSpec((NC, BM, NFEAT), lambda i: (0, i, 0)),
                  pl.BlockSpec((NFEAT, NFEAT), lambda i: (0, 0))],
        out_specs=pl.BlockSpec((BM, NFEAT), lambda i: (i, 0)),
        out_shape=jax.ShapeDtypeStruct((N_NODES, NFEAT), jnp.float32),
    )(P, W)


def _mm_uv(Q, W2, W3):
    def body(q_ref, w2_ref, w3_ref, o_ref):
        w3r = jnp.concatenate([w3_ref[0:NFEAT, :], w3_ref[NFEAT:, :]], axis=1)
        w23 = jnp.dot(w2_ref[...], w3r, preferred_element_type=jnp.float32)
        s = q_ref[0] + q_ref[1]
        o_ref[...] = jnp.dot(s, w23, preferred_element_type=jnp.float32)

    return pl.pallas_call(
        body,
        grid=(N_NODES // BM,),
        in_specs=[pl.BlockSpec((NC, BM, NFEAT), lambda i: (0, i, 0)),
                  pl.BlockSpec((NFEAT, NFEAT), lambda i: (0, 0)),
                  pl.BlockSpec((2 * NFEAT, 2), lambda i: (0, 0))],
        out_specs=pl.BlockSpec((BM, 4), lambda i: (i, 0)),
        out_shape=jax.ShapeDtypeStruct((N_NODES, 4), jnp.float32),
    )(Q, W2, W3)


def _decode(uvf, pq):
    C2 = 2 * C
    LAST = CHUNKS_PER_W - 1

    @functools.partial(
        pl.kernel,
        out_type=(jax.ShapeDtypeStruct((N_EDGES,), jnp.float32),
                  jax.ShapeDtypeStruct((N_EDGES,), jnp.float32)),
        mesh=_sc_mesh(),
        scratch_types=[
            pltpu.VMEM((N_NODES * 4,), jnp.float32),
            pltpu.VMEM((2 * C2,), jnp.int32),
            pltpu.VMEM((2, 2, C), jnp.float32),
            pltpu.SemaphoreType.DMA,
            pltpu.SemaphoreType.DMA,
            pltpu.SemaphoreType.DMA,
            pltpu.SemaphoreType.DMA,
        ],
        compiler_params=pltpu.CompilerParams(needs_layout_passes=False),
    )
    def dec(uv_hbm, pq_hbm, o0_hbm, o1_hbm, uv_v, pq2, ob,
            semI0, semI1, semO0, semO1):
        cid = lax.axis_index("c")
        sid = lax.axis_index("s")
        wid = cid * NS + sid
        semI = (semI0, semI1)
        semO = (semO0, semO1)
        pltpu.sync_copy(uv_hbm, uv_v)

        def issue_idx(j, s):
            g = j * NW + wid
            pltpu.async_copy(pq_hbm.at[pl.ds(g * C2, C2)],
                             pq2.at[pl.ds(s * C2, C2)], semI[s])

        def wait_idx(s):
            pltpu.make_async_copy(pq_hbm.at[pl.ds(0, C2)],
                                  pq2.at[pl.ds(s * C2, C2)], semI[s]).wait()

        def issue_out(j, s):
            base = (j * NW + wid) * C
            pltpu.async_copy(ob.at[s, 0], o0_hbm.at[pl.ds(base, C)], semO[s])
            pltpu.async_copy(ob.at[s, 1], o1_hbm.at[pl.ds(base, C)], semO[s])

        def wait_out(s):
            pltpu.make_async_copy(ob.at[s, 0], o0_hbm.at[pl.ds(0, C)],
                                  semO[s]).wait()
            pltpu.make_async_copy(ob.at[s, 1], o1_hbm.at[pl.ds(0, C)],
                                  semO[s]).wait()

        def step(j, c, wait_old=True, prefetch=True):
            s = c % 2
            if wait_old:
                wait_out(s)
            wait_idx(s)
            for g in range(C // L):
                si = pq2[pl.ds(s * C2 + g * L, L)] * 4
                di = pq2[pl.ds(s * C2 + C + g * L, L)] * 4
                u0 = plsc.load_gather(uv_v, [si])
                u1 = plsc.load_gather(uv_v, [si + 1])
                v0 = plsc.load_gather(uv_v, [di + 2])
                v1 = plsc.load_gather(uv_v, [di + 3])
                ob[s, 0, pl.ds(g * L, L)] = u0 + v0
                ob[s, 1, pl.ds(g * L, L)] = u1 + v1
            issue_out(j, s)
            if prefetch:
                issue_idx(j + 2, s)

        issue_idx(0, 0)
        issue_idx(1, 1)
        step(0, 0, wait_old=False)
        step(1, 1, wait_old=False)

        def duo(jj, carry):
            j0 = 2 * jj
            step(j0 + 0, 0)
            step(j0 + 1, 1)
            return carry
        lax.fori_loop(1, 61, duo, 0)

        step(122, 0)
        step(123, 1, prefetch=False)
        step(124, 0, prefetch=False)
        wait_out(1)
        wait_out(0)

    o0, o1 = dec(uvf, pq)
    return jnp.stack([o0, o1], axis=1)


def kernel(x, edge_index1, edge_index2, edge_weight1, edge_weight2,
           pos_edge_index, W1, W2, W3):
    sw1, dst1 = _pack_edges(edge_index1, edge_weight1)
    sw2_, dst2 = _pack_edges(edge_index2, edge_weight2)
    pq = jnp.concatenate([pos_edge_index[0].astype(jnp.int32).reshape(-1, C),
                          pos_edge_index[1].astype(jnp.int32).reshape(-1, C)],
                         axis=1).reshape(-1)

    P1 = _edge_aggregate(x.astype(jnp.float32), sw1, dst1)
    h = _mm_relu(P1, W1)
    P2 = _edge_aggregate(h, sw2_, dst2)
    uv = _mm_uv(P2, W2, W3)
    return _decode(uv.reshape(-1), pq)


# speedup vs baseline: 10.5616x; 1.0445x over previous
import functools

import jax
import jax.numpy as jnp
from jax import lax
from jax.experimental import pallas as pl
from jax.experimental.pallas import tpu as pltpu
from jax.experimental.pallas import tpu_sc as plsc

N_NODES = 10000
N_EDGES = 320000
NFEAT = 128

NC, NS, L = 2, 16, 16
NW = NC * NS
C = 80
CHUNKS_PER_W = N_EDGES // (NW * C)
RBLK = 80
NBLK = N_NODES // RBLK
GROUPS = NFEAT // L

ET = N_EDGES // NW
CA = 128
CA2 = 2 * CA
CPT = ET // CA
TAIL = ET - CPT * CA
SWPT = CPT * CA2 + 2 * TAIL

BM = 1000


def _sc_mesh():
    return plsc.VectorSubcoreMesh(core_axis_name="c", subcore_axis_name="s")


def _pack_edges(edge_index, w):
    src = edge_index[0].astype(jnp.int32).reshape(NW, ET)
    wbits = jax.lax.bitcast_convert_type(w.astype(jnp.float32),
                                         jnp.int32).reshape(NW, ET)
    body = jnp.concatenate([src[:, :CPT * CA].reshape(NW, CPT, CA),
                            wbits[:, :CPT * CA].reshape(NW, CPT, CA)],
                           axis=2).reshape(NW, CPT * CA2)
    tail = jnp.concatenate([src[:, CPT * CA:], wbits[:, CPT * CA:]], axis=1)
    sw = jnp.concatenate([body, tail], axis=1).reshape(-1)
    return sw, edge_index[1].astype(jnp.int32)


def _edge_aggregate(table, sw, dstf):
    @functools.partial(
        pl.kernel,
        out_type=jax.ShapeDtypeStruct((NC, N_NODES, NFEAT), jnp.float32),
        mesh=_sc_mesh(),
        scratch_types=[
            pltpu.VMEM((2 * CA2,), jnp.int32),
            pltpu.VMEM((4, CA), jnp.int32),
            pltpu.VMEM((TAIL,), jnp.int32),
            pltpu.VMEM((2, CA, NFEAT), jnp.float32),
            pltpu.VMEM_SHARED((N_NODES, NFEAT), jnp.float32),
            pltpu.SemaphoreType.DMA,
            pltpu.SemaphoreType.DMA,
            pltpu.SemaphoreType.DMA,
            pltpu.SemaphoreType.DMA,
            pltpu.SemaphoreType.DMA,
            pltpu.SemaphoreType.DMA,
            pltpu.SemaphoreType.DMA,
        ],
        compiler_params=pltpu.CompilerParams(needs_layout_passes=False),
    )
    def agg(table_hbm, sw_hbm, dst_hbm, out_hbm,
            sw2, dst4, dstT, rows2, acc,
            semI0, semI1, semG0, semG1, semS0, semS1, semT):
        cid = lax.axis_index("c")
        sid = lax.axis_index("s")
        wid = cid * NS + sid
        semI = (semI0, semI1)
        semG = (semG0, semG1)
        semS = (semS0, semS1)

        def zfill(i, carry):
            for g in range(GROUPS):
                rows2[0, i, pl.ds(g * L, L)] = jnp.zeros((L,), jnp.float32)
            return carry
        lax.fori_loop(0, RBLK, zfill, 0)
        for k in range(NBLK // NS + 1):
            b = sid + NS * k
            @pl.when(b < NBLK)
            def _():
                pltpu.sync_copy(rows2.at[0, pl.ds(0, RBLK)],
                                acc.at[pl.ds(b * RBLK, RBLK)])
        plsc.subcore_barrier()

        def issue_idx(j, s, d):
            pltpu.async_copy(sw_hbm.at[pl.ds(wid * SWPT + j * CA2, CA2)],
                             sw2.at[pl.ds(s * CA2, CA2)], semI[s])
            pltpu.async_copy(dst_hbm.at[pl.ds(wid * ET + j * CA, CA)],
                             dst4.at[d], semI[s])

        def wait_idx(s, d):
            pltpu.make_async_copy(sw_hbm.at[pl.ds(0, CA2)],
                                  sw2.at[pl.ds(s * CA2, CA2)], semI[s]).wait()
            pltpu.make_async_copy(dst_hbm.at[pl.ds(0, CA)], dst4.at[d],
                                  semI[s]).wait()

        def issue_gather(s):
            pltpu.async_copy(table_hbm.at[sw2.at[pl.ds(s * CA2, CA)]],
                             rows2.at[s], semG[s])

        def wait_gather(s):
            pltpu.make_async_copy(table_hbm.at[sw2.at[pl.ds(s * CA2, CA)]],
                                  rows2.at[s], semG[s]).wait()

        def scale(s):
            def body(e, carry):
                wb = plsc.bitcast(
                    plsc.load_gather(
                        sw2, [jnp.full((L,), s * CA2 + CA, jnp.int32) + e]),
                    jnp.float32)
                for g in range(GROUPS):
                    rows2[s, e, pl.ds(g * L, L)] = rows2[s, e, pl.ds(g * L, L)] * wb
                return carry
            lax.fori_loop(0, CA, body, 0, unroll=4)

        def issue_scatter(s, d):
            pltpu.async_copy(rows2.at[s], acc.at[dst4.at[d]], semS[s], add=True)

        def wait_scatter(s, d):
            pltpu.make_async_copy(rows2.at[s], acc.at[dst4.at[d]],
                                  semS[s]).wait()

        def step(j, c, first=False, prefetch=True, fetch_next=True):
            s, o, d = c % 2, 1 - c % 2, c % 4
            if not first:
                wait_scatter(o, (c - 1) % 4)
            if fetch_next:
                wait_idx(o, (c + 1) % 4)
                issue_gather(o)
            wait_gather(s)
            scale(s)
            issue_scatter(s, d)
            if prefetch:
                issue_idx(j + 2, s, (c + 2) % 4)

        issue_idx(0, 0, 0)
        issue_idx(1, 1, 1)
        wait_idx(0, 0)
        issue_gather(0)
        step(0, 0, first=True)
        step(1, 1)
        step(2, 2)
        step(3, 3)

        def quad(jj, carry):
            j0 = 4 * jj
            step(j0 + 0, 0)
            step(j0 + 1, 1)
            step(j0 + 2, 2)
            step(j0 + 3, 3)
            return carry
        lax.fori_loop(1, (CPT - 6) // 4, quad, 0)

        step(72, 0)
        step(73, 1)
        step(74, 2)
        step(75, 3)
        step(76, 0, prefetch=False)
        step(77, 1, prefetch=False, fetch_next=False)
        wait_scatter(1, 1)

        pltpu.sync_copy(sw_hbm.at[pl.ds(wid * SWPT + CPT * CA2, 2 * TAIL)],
                        sw2.at[pl.ds(0, 2 * TAIL)])
        pltpu.sync_copy(dst_hbm.at[pl.ds(wid * ET + CPT * CA, TAIL)], dstT)
        pltpu.async_copy(table_hbm.at[sw2.at[pl.ds(0, TAIL)]],
                         rows2.at[0, pl.ds(0, TAIL)], semT).wait()

        def tbody(e, carry):
            wb = plsc.bitcast(
                plsc.load_gather(sw2, [jnp.full((L,), TAIL, jnp.int32) + e]),
                jnp.float32)
            for g in range(GROUPS):
                rows2[0, e, pl.ds(g * L, L)] = rows2[0, e, pl.ds(g * L, L)] * wb
            return carry
        lax.fori_loop(0, TAIL, tbody, 0, unroll=4)
        pltpu.sync_copy(rows2.at[0, pl.ds(0, TAIL)], acc.at[dstT], add=True)

        plsc.subcore_barrier()
        for k in range(NBLK // NS + 1):
            b = sid + NS * k
            @pl.when(b < NBLK)
            def _():
                r0 = b * RBLK
                pltpu.sync_copy(acc.at[pl.ds(r0, RBLK)],
                                out_hbm.at[cid, pl.ds(r0, RBLK)])

    return agg(table, sw, dstf)


def _mm_relu(P, W):
    def body(p_ref, w_ref, o_ref):
        s = p_ref[0] + p_ref[1]
        o_ref[...] = jnp.maximum(
            jnp.dot(s, w_ref[...], preferred_element_type=jnp.float32), 0.0)

    return pl.pallas_call(
        body,
        grid=(N_NODES // BM,),
        in_specs=[pl.BlockSpec((NC, BM, NFEAT), lambda i: (0, i, 0)),
                  pl.BlockSpec((NFEAT, NFEAT), lambda i: (0, 0))],
        out_specs=pl.BlockSpec((BM, NFEAT), lambda i: (i, 0)),
        out_shape=jax.ShapeDtypeStruct((N_NODES, NFEAT), jnp.float32),
    )(P, W)


def _mm_uv(Q, W2, W3):
    def body(q_ref, w2_ref, w3_ref, o_ref):
        w3r = jnp.concatenate([w3_ref[0:NFEAT, :], w3_ref[NFEAT:, :]], axis=1)
        w23 = jnp.dot(w2_ref[...], w3r, preferred_element_type=jnp.float32)
        s = q_ref[0] + q_ref[1]
        o_ref[...] = jnp.dot(s, w23, preferred_element_type=jnp.float32)

    return pl.pallas_call(
        body,
        grid=(N_NODES // BM,),
        in_specs=[pl.BlockSpec((NC, BM, NFEAT), lambda i: (0, i, 0)),
                  pl.BlockSpec((NFEAT, NFEAT), lambda i: (0, 0)),
                  pl.BlockSpec((2 * NFEAT, 2), lambda i: (0, 0))],
        out_specs=pl.BlockSpec((BM, 4), lambda i: (i, 0)),
        out_shape=jax.ShapeDtypeStruct((N_NODES, 4), jnp.float32),
    )(Q, W2, W3)


def _decode(uvf, pq):
    C2 = 2 * C
    LAST = CHUNKS_PER_W - 1

    @functools.partial(
        pl.kernel,
        out_type=(jax.ShapeDtypeStruct((N_EDGES,), jnp.float32),
                  jax.ShapeDtypeStruct((N_EDGES,), jnp.float32)),
        mesh=_sc_mesh(),
        scratch_types=[
            pltpu.VMEM((N_NODES * 4,), jnp.float32),
            pltpu.VMEM((2 * C2,), jnp.int32),
            pltpu.VMEM((2, 2, C), jnp.float32),
            pltpu.SemaphoreType.DMA,
            pltpu.SemaphoreType.DMA,
            pltpu.SemaphoreType.DMA,
            pltpu.SemaphoreType.DMA,
        ],
        compiler_params=pltpu.CompilerParams(needs_layout_passes=False),
    )
    def dec(uv_hbm, pq_hbm, o0_hbm, o1_hbm, uv_v, pq2, ob,
            semI0, semI1, semO0, semO1):
        cid = lax.axis_index("c")
        sid = lax.axis_index("s")
        wid = cid * NS + sid
        semI = (semI0, semI1)
        semO = (semO0, semO1)
        pltpu.sync_copy(uv_hbm, uv_v)

        def issue_idx(j, s):
            g = j * NW + wid
            pltpu.async_copy(pq_hbm.at[pl.ds(g * C2, C2)],
                             pq2.at[pl.ds(s * C2, C2)], semI[s])

        def wait_idx(s):
            pltpu.make_async_copy(pq_hbm.at[pl.ds(0, C2)],
                                  pq2.at[pl.ds(s * C2, C2)], semI[s]).wait()

        def issue_out(j, s):
            base = (j * NW + wid) * C
            pltpu.async_copy(ob.at[s, 0], o0_hbm.at[pl.ds(base, C)], semO[s])
            pltpu.async_copy(ob.at[s, 1], o1_hbm.at[pl.ds(base, C)], semO[s])

        def wait_out(s):
            pltpu.make_async_copy(ob.at[s, 0], o0_hbm.at[pl.ds(0, C)],
                                  semO[s]).wait()
            pltpu.make_async_copy(ob.at[s, 1], o1_hbm.at[pl.ds(0, C)],
                                  semO[s]).wait()

        def step(j, c, wait_old=True, prefetch=True):
            s = c % 2
            if wait_old:
                wait_out(s)
            wait_idx(s)
            for g in range(C // L):
                si = pq2[pl.ds(s * C2 + g * L, L)] * 4
                di = pq2[pl.ds(s * C2 + C + g * L, L)] * 4
                u0 = plsc.load_gather(uv_v, [si])
                u1 = plsc.load_gather(uv_v, [si + 1])
                v0 = plsc.load_gather(uv_v, [di + 2])
                v1 = plsc.load_gather(uv_v, [di + 3])
                ob[s, 0, pl.ds(g * L, L)] = u0 + v0
                ob[s, 1, pl.ds(g * L, L)] = u1 + v1
            issue_out(j, s)
            if prefetch:
                issue_idx(j + 2, s)

        issue_idx(0, 0)
        issue_idx(1, 1)
        step(0, 0, wait_old=False)
        step(1, 1, wait_old=False)

        def duo(jj, carry):
            j0 = 2 * jj
            step(j0 + 0, 0)
            step(j0 + 1, 1)
            return carry
        lax.fori_loop(1, 61, duo, 0)

        step(122, 0)
        step(123, 1, prefetch=False)
        step(124, 0, prefetch=False)
        wait_out(1)
        wait_out(0)

    o0, o1 = dec(uvf, pq)
    return jnp.stack([o0, o1], axis=1)


def kernel(x, edge_index1, edge_index2, edge_weight1, edge_weight2,
           pos_edge_index, W1, W2, W3):
    sw1, dst1 = _pack_edges(edge_index1, edge_weight1)
    sw2_, dst2 = _pack_edges(edge_index2, edge_weight2)
    pq = jnp.concatenate([pos_edge_index[0].astype(jnp.int32).reshape(-1, C),
                          pos_edge_index[1].astype(jnp.int32).reshape(-1, C)],
                         axis=1).reshape(-1)

    P1 = _edge_aggregate(x.astype(jnp.float32), sw1, dst1)
    h = _mm_relu(P1, W1)
    P2 = _edge_aggregate(h, sw2_, dst2)
    uv = _mm_uv(P2, W2, W3)
    return _decode(uv.reshape(-1), pq)


# speedup vs baseline: 10.5675x; 1.0006x over previous
import functools

import jax
import jax.numpy as jnp
from jax import lax
from jax.experimental import pallas as pl
from jax.experimental.pallas import tpu as pltpu
from jax.experimental.pallas import tpu_sc as plsc

N_NODES = 10000
N_EDGES = 320000
NFEAT = 128

NC, NS, L = 2, 16, 16
NW = NC * NS
C = 80
CHUNKS_PER_W = N_EDGES // (NW * C)
RBLK = 80
NBLK = N_NODES // RBLK
GROUPS = NFEAT // L

ET = N_EDGES // NW
CA = 128
CA2 = 2 * CA
CPT = ET // CA
TAIL = ET - CPT * CA
SWPT = CPT * CA2 + 2 * TAIL

BM = 1000


def _sc_mesh():
    return plsc.VectorSubcoreMesh(core_axis_name="c", subcore_axis_name="s")


def _pack_edges(edge_index, w):
    src = edge_index[0].astype(jnp.int32).reshape(NW, ET)
    wbits = jax.lax.bitcast_convert_type(w.astype(jnp.float32),
                                         jnp.int32).reshape(NW, ET)
    body = jnp.concatenate([src[:, :CPT * CA].reshape(NW, CPT, CA),
                            wbits[:, :CPT * CA].reshape(NW, CPT, CA)],
                           axis=2).reshape(NW, CPT * CA2)
    tail = jnp.concatenate([src[:, CPT * CA:], wbits[:, CPT * CA:]], axis=1)
    sw = jnp.concatenate([body, tail], axis=1).reshape(-1)
    return sw, edge_index[1].astype(jnp.int32)


def _edge_aggregate(table, sw, dstf):
    @functools.partial(
        pl.kernel,
        out_type=jax.ShapeDtypeStruct((NC, N_NODES, NFEAT), jnp.float32),
        mesh=_sc_mesh(),
        scratch_types=[
            pltpu.VMEM((2 * CA2,), jnp.int32),
            pltpu.VMEM((4, CA), jnp.int32),
            pltpu.VMEM((TAIL,), jnp.int32),
            pltpu.VMEM((2, CA, NFEAT), jnp.float32),
            pltpu.VMEM_SHARED((N_NODES, NFEAT), jnp.float32),
            pltpu.SemaphoreType.DMA,
            pltpu.SemaphoreType.DMA,
            pltpu.SemaphoreType.DMA,
            pltpu.SemaphoreType.DMA,
            pltpu.SemaphoreType.DMA,
            pltpu.SemaphoreType.DMA,
            pltpu.SemaphoreType.DMA,
        ],
        compiler_params=pltpu.CompilerParams(needs_layout_passes=False),
    )
    def agg(table_hbm, sw_hbm, dst_hbm, out_hbm,
            sw2, dst4, dstT, rows2, acc,
            semI0, semI1, semG0, semG1, semS0, semS1, semT):
        cid = lax.axis_index("c")
        sid = lax.axis_index("s")
        wid = cid * NS + sid
        semI = (semI0, semI1)
        semG = (semG0, semG1)
        semS = (semS0, semS1)


        def issue_idx(j, s, d):
            pltpu.async_copy(sw_hbm.at[pl.ds(wid * SWPT + j * CA2, CA2)],
                             sw2.at[pl.ds(s * CA2, CA2)], semI[s])
            pltpu.async_copy(dst_hbm.at[pl.ds(wid * ET + j * CA, CA)],
                             dst4.at[d], semI[s])

        def wait_idx(s, d):
            pltpu.make_async_copy(sw_hbm.at[pl.ds(0, CA2)],
                                  sw2.at[pl.ds(s * CA2, CA2)], semI[s]).wait()
            pltpu.make_async_copy(dst_hbm.at[pl.ds(0, CA)], dst4.at[d],
                                  semI[s]).wait()

        def issue_gather(s):
            pltpu.async_copy(table_hbm.at[sw2.at[pl.ds(s * CA2, CA)]],
                             rows2.at[s], semG[s])

        def wait_gather(s):
            pltpu.make_async_copy(table_hbm.at[sw2.at[pl.ds(s * CA2, CA)]],
                                  rows2.at[s], semG[s]).wait()

        def scale(s):
            def body(e, carry):
                wb = plsc.bitcast(
                    plsc.load_gather(
                        sw2, [jnp.full((L,), s * CA2 + CA, jnp.int32) + e]),
                    jnp.float32)
                for g in range(GROUPS):
                    rows2[s, e, pl.ds(g * L, L)] = rows2[s, e, pl.ds(g * L, L)] * wb
                return carry
            lax.fori_loop(0, CA, body, 0, unroll=8)

        def issue_scatter(s, d):
            pltpu.async_copy(rows2.at[s], acc.at[dst4.at[d]], semS[s], add=True)

        def wait_scatter(s, d):
            pltpu.make_async_copy(rows2.at[s], acc.at[dst4.at[d]],
                                  semS[s]).wait()

        def step(j, c, first=False, prefetch=True, fetch_next=True):
            s, o, d = c % 2, 1 - c % 2, c % 4
            if not first:
                wait_scatter(o, (c - 1) % 4)
            if fetch_next:
                wait_idx(o, (c + 1) % 4)
                issue_gather(o)
            wait_gather(s)
            scale(s)
            issue_scatter(s, d)
            if prefetch:
                issue_idx(j + 2, s, (c + 2) % 4)

        issue_idx(0, 0, 0)
        issue_idx(1, 1, 1)
        wait_idx(0, 0)
        issue_gather(0)

        def zfill(i, carry):
            for g in range(GROUPS):
                rows2[1, i, pl.ds(g * L, L)] = jnp.zeros((L,), jnp.float32)
            return carry
        lax.fori_loop(0, RBLK, zfill, 0, unroll=4)
        for k in range(NBLK // NS + 1):
            b = sid + NS * k
            @pl.when(b < NBLK)
            def _():
                pltpu.sync_copy(rows2.at[1, pl.ds(0, RBLK)],
                                acc.at[pl.ds(b * RBLK, RBLK)])
        plsc.subcore_barrier()

        step(0, 0, first=True)
        step(1, 1)
        step(2, 2)
        step(3, 3)

        def quad(jj, carry):
            j0 = 4 * jj
            step(j0 + 0, 0)
            step(j0 + 1, 1)
            step(j0 + 2, 2)
            step(j0 + 3, 3)
            return carry
        lax.fori_loop(1, (CPT - 6) // 4, quad, 0)

        step(72, 0)
        step(73, 1)
        step(74, 2)
        step(75, 3)
        step(76, 0, prefetch=False)
        step(77, 1, prefetch=False, fetch_next=False)
        wait_scatter(1, 1)

        pltpu.sync_copy(sw_hbm.at[pl.ds(wid * SWPT + CPT * CA2, 2 * TAIL)],
                        sw2.at[pl.ds(0, 2 * TAIL)])
        pltpu.sync_copy(dst_hbm.at[pl.ds(wid * ET + CPT * CA, TAIL)], dstT)
        pltpu.async_copy(table_hbm.at[sw2.at[pl.ds(0, TAIL)]],
                         rows2.at[0, pl.ds(0, TAIL)], semT).wait()

        def tbody(e, carry):
            wb = plsc.bitcast(
                plsc.load_gather(sw2, [jnp.full((L,), TAIL, jnp.int32) + e]),
                jnp.float32)
            for g in range(GROUPS):
                rows2[0, e, pl.ds(g * L, L)] = rows2[0, e, pl.ds(g * L, L)] * wb
            return carry
        lax.fori_loop(0, TAIL, tbody, 0, unroll=4)
        pltpu.sync_copy(rows2.at[0, pl.ds(0, TAIL)], acc.at[dstT], add=True)

        plsc.subcore_barrier()
        for k in range(NBLK // NS + 1):
            b = sid + NS * k
            @pl.when(b < NBLK)
            def _():
                r0 = b * RBLK
                pltpu.sync_copy(acc.at[pl.ds(r0, RBLK)],
                                out_hbm.at[cid, pl.ds(r0, RBLK)])

    return agg(table, sw, dstf)


def _mm_relu(P, W):
    def body(p_ref, w_ref, o_ref):
        s = p_ref[0] + p_ref[1]
        o_ref[...] = jnp.maximum(
            jnp.dot(s, w_ref[...], preferred_element_type=jnp.float32), 0.0)

    return pl.pallas_call(
        body,
        grid=(N_NODES // BM,),
        in_specs=[pl.BlockSpec((NC, BM, NFEAT), lambda i: (0, i, 0)),
                  pl.BlockSpec((NFEAT, NFEAT), lambda i: (0, 0))],
        out_specs=pl.BlockSpec((BM, NFEAT), lambda i: (i, 0)),
        out_shape=jax.ShapeDtypeStruct((N_NODES, NFEAT), jnp.float32),
    )(P, W)


def _mm_uv(Q, W2, W3):
    def body(q_ref, w2_ref, w3_ref, o_ref):
        w3r = jnp.concatenate([w3_ref[0:NFEAT, :], w3_ref[NFEAT:, :]], axis=1)
        w23 = jnp.dot(w2_ref[...], w3r, preferred_element_type=jnp.float32)
        s = q_ref[0] + q_ref[1]
        o_ref[...] = jnp.dot(s, w23, preferred_element_type=jnp.float32)

    return pl.pallas_call(
        body,
        grid=(N_NODES // BM,),
        in_specs=[pl.BlockSpec((NC, BM, NFEAT), lambda i: (0, i, 0)),
                  pl.BlockSpec((NFEAT, NFEAT), lambda i: (0, 0)),
                  pl.BlockSpec((2 * NFEAT, 2), lambda i: (0, 0))],
        out_specs=pl.BlockSpec((BM, 4), lambda i: (i, 0)),
        out_shape=jax.ShapeDtypeStruct((N_NODES, 4), jnp.float32),
    )(Q, W2, W3)


def _decode(uvf, pq):
    C2 = 2 * C
    LAST = CHUNKS_PER_W - 1

    @functools.partial(
        pl.kernel,
        out_type=(jax.ShapeDtypeStruct((N_EDGES,), jnp.float32),
                  jax.ShapeDtypeStruct((N_EDGES,), jnp.float32)),
        mesh=_sc_mesh(),
        scratch_types=[
            pltpu.VMEM((N_NODES * 4,), jnp.float32),
            pltpu.VMEM((2 * C2,), jnp.int32),
            pltpu.VMEM((2, 2, C), jnp.float32),
            pltpu.SemaphoreType.DMA,
            pltpu.SemaphoreType.DMA,
            pltpu.SemaphoreType.DMA,
            pltpu.SemaphoreType.DMA,
        ],
        compiler_params=pltpu.CompilerParams(needs_layout_passes=False),
    )
    def dec(uv_hbm, pq_hbm, o0_hbm, o1_hbm, uv_v, pq2, ob,
            semI0, semI1, semO0, semO1):
        cid = lax.axis_index("c")
        sid = lax.axis_index("s")
        wid = cid * NS + sid
        semI = (semI0, semI1)
        semO = (semO0, semO1)
        pltpu.sync_copy(uv_hbm, uv_v)

        def issue_idx(j, s):
            g = j * NW + wid
            pltpu.async_copy(pq_hbm.at[pl.ds(g * C2, C2)],
                             pq2.at[pl.ds(s * C2, C2)], semI[s])

        def wait_idx(s):
            pltpu.make_async_copy(pq_hbm.at[pl.ds(0, C2)],
                                  pq2.at[pl.ds(s * C2, C2)], semI[s]).wait()

        def issue_out(j, s):
            base = (j * NW + wid) * C
            pltpu.async_copy(ob.at[s, 0], o0_hbm.at[pl.ds(base, C)], semO[s])
            pltpu.async_copy(ob.at[s, 1], o1_hbm.at[pl.ds(base, C)], semO[s])

        def wait_out(s):
            pltpu.make_async_copy(ob.at[s, 0], o0_hbm.at[pl.ds(0, C)],
                                  semO[s]).wait()
            pltpu.make_async_copy(ob.at[s, 1], o1_hbm.at[pl.ds(0, C)],
                                  semO[s]).wait()

        def step(j, c, wait_old=True, prefetch=True):
            s = c % 2
            if wait_old:
                wait_out(s)
            wait_idx(s)
            for g in range(C // L):
                si = pq2[pl.ds(s * C2 + g * L, L)] * 4
                di = pq2[pl.ds(s * C2 + C + g * L, L)] * 4
                u0 = plsc.load_gather(uv_v, [si])
                u1 = plsc.load_gather(uv_v, [si + 1])
                v0 = plsc.load_gather(uv_v, [di + 2])
                v1 = plsc.load_gather(uv_v, [di + 3])
                ob[s, 0, pl.ds(g * L, L)] = u0 + v0
                ob[s, 1, pl.ds(g * L, L)] = u1 + v1
            issue_out(j, s)
            if prefetch:
                issue_idx(j + 2, s)

        issue_idx(0, 0)
        issue_idx(1, 1)
        step(0, 0, wait_old=False)
        step(1, 1, wait_old=False)

        def duo(jj, carry):
            j0 = 2 * jj
            step(j0 + 0, 0)
            step(j0 + 1, 1)
            return carry
        lax.fori_loop(1, 61, duo, 0)

        step(122, 0)
        step(123, 1, prefetch=False)
        step(124, 0, prefetch=False)
        wait_out(1)
        wait_out(0)

    o0, o1 = dec(uvf, pq)
    return jnp.stack([o0, o1], axis=1)


def kernel(x, edge_index1, edge_index2, edge_weight1, edge_weight2,
           pos_edge_index, W1, W2, W3):
    sw1, dst1 = _pack_edges(edge_index1, edge_weight1)
    sw2_, dst2 = _pack_edges(edge_index2, edge_weight2)
    pq = jnp.concatenate([pos_edge_index[0].astype(jnp.int32).reshape(-1, C),
                          pos_edge_index[1].astype(jnp.int32).reshape(-1, C)],
                         axis=1).reshape(-1)

    P1 = _edge_aggregate(x.astype(jnp.float32), sw1, dst1)
    h = _mm_relu(P1, W1)
    P2 = _edge_aggregate(h, sw2_, dst2)
    uv = _mm_uv(P2, W2, W3)
    return _decode(uv.reshape(-1), pq)
